# Initial kernel scaffold; baseline (speedup 1.0000x reference)
import functools

import jax
import jax.numpy as jnp
from jax import lax
from jax.experimental import pallas as pl
from jax.experimental.pallas import tpu as pltpu
from jax.experimental.pallas import tpu_sc as plsc

_NUM_VOCAB = 100000
_MAXLEN = 200
_HID = 64
_BATCH = 4096
_SEQ = 200

_NC = 2
_NS = 16
_NW = _NC * _NS
_SEQ_PER_W = _BATCH // _NW

_GCHUNK = 100
_NG = _SEQ // _GCHUNK


def _body(x_hbm, emb_hbm, pos_hbm, out_hbm, idx_v, rows_v, pos_v, sem):
    wid = lax.axis_index("s") * _NC + lax.axis_index("c")
    pltpu.sync_copy(pos_hbm, pos_v)

    def step(i, carry):
        seq = wid * _SEQ_PER_W + i
        pltpu.sync_copy(x_hbm.at[seq], idx_v)
        pltpu.sync_copy(pos_v, rows_v)
        for j in range(_NG):
            pltpu.async_copy(
                emb_hbm.at[idx_v.at[j]],
                rows_v.at[pl.ds(j * _GCHUNK, _GCHUNK)],
                sem,
                add=True,
            )
        for j in range(_NG):
            pltpu.make_async_copy(
                emb_hbm.at[idx_v.at[j]],
                rows_v.at[pl.ds(j * _GCHUNK, _GCHUNK)],
                sem,
            ).wait()
        pltpu.sync_copy(rows_v, out_hbm.at[seq])
        return carry

    lax.fori_loop(0, _SEQ_PER_W, step, 0)


@jax.jit
def _embed(x, emb, pos_emb):
    mesh = plsc.VectorSubcoreMesh(core_axis_name="c", subcore_axis_name="s")
    k = pl.kernel(
        _body,
        out_type=jax.ShapeDtypeStruct((_BATCH, _SEQ, _HID), jnp.float32),
        mesh=mesh,
        scratch_types=[
            pltpu.VMEM((_NG, _GCHUNK), jnp.int32),
            pltpu.VMEM((_SEQ, _HID), jnp.float32),
            pltpu.VMEM((_MAXLEN, _HID), jnp.float32),
            pltpu.SemaphoreType.DMA,
        ],
    )
    xr = x.reshape(_BATCH, _NG, _GCHUNK).astype(jnp.int32)
    return k(xr, emb, pos_emb)


def kernel(x, emb, pos_emb):
    return _embed(x, emb, pos_emb)


# speedup vs baseline: 2.4840x; 2.4840x over previous
import functools

import jax
import jax.numpy as jnp
from jax import lax
from jax.experimental import pallas as pl
from jax.experimental.pallas import tpu as pltpu
from jax.experimental.pallas import tpu_sc as plsc

_NUM_VOCAB = 100000
_MAXLEN = 200
_HID = 64
_BATCH = 4096
_SEQ = 200

_NC = 2
_NS = 16
_NW = _NC * _NS
_SEQ_PER_W = _BATCH // _NW

_GCHUNK = 100
_NG = _SEQ // _GCHUNK


def _body(x_hbm, emb_hbm, pos_hbm, out_hbm, idx_v, rows_v, sem):
    wid = lax.axis_index("s") * _NC + lax.axis_index("c")

    def step(i, carry):
        seq = wid * _SEQ_PER_W + i
        pltpu.sync_copy(x_hbm.at[seq], idx_v)
        pltpu.sync_copy(pos_hbm, rows_v)
        for j in range(_NG):
            pltpu.async_copy(
                emb_hbm.at[idx_v.at[j]],
                rows_v.at[pl.ds(j * _GCHUNK, _GCHUNK)],
                sem,
                add=True,
            )
        for j in range(_NG):
            pltpu.make_async_copy(
                emb_hbm.at[idx_v.at[j]],
                rows_v.at[pl.ds(j * _GCHUNK, _GCHUNK)],
                sem,
            ).wait()
        pltpu.sync_copy(rows_v, out_hbm.at[seq])
        return carry

    lax.fori_loop(0, _SEQ_PER_W, step, 0)


@jax.jit
def _embed(x, emb, pos_emb):
    mesh = plsc.VectorSubcoreMesh(core_axis_name="c", subcore_axis_name="s")
    k = pl.kernel(
        _body,
        out_type=jax.ShapeDtypeStruct((_BATCH, _SEQ, _HID), jnp.float32),
        mesh=mesh,
        scratch_types=[
            pltpu.VMEM((_NG, _GCHUNK), jnp.int32),
            pltpu.VMEM((_SEQ, _HID), jnp.float32),
            pltpu.SemaphoreType.DMA,
        ],
        compiler_params=pltpu.CompilerParams(use_tc_tiling_on_sc=False),
    )
    xr = x.reshape(_BATCH, _NG, _GCHUNK).astype(jnp.int32)
    return k(xr, emb, pos_emb)


def kernel(x, emb, pos_emb):
    return _embed(x, emb, pos_emb)


# speedup vs baseline: 2.8505x; 1.1476x over previous
import functools

import jax
import jax.numpy as jnp
from jax import lax
from jax.experimental import pallas as pl
from jax.experimental.pallas import tpu as pltpu
from jax.experimental.pallas import tpu_sc as plsc

_NUM_VOCAB = 100000
_MAXLEN = 200
_HID = 64
_BATCH = 4096
_SEQ = 200

_NC = 2
_NS = 16
_NW = _NC * _NS

_GCHUNK = 100
_NG = _SEQ // _GCHUNK

_CS = 2
_NBUF = 4
_CHUNKS = _BATCH // (_NW * _CS)
_ROWS = _CS * _SEQ


def _body(x_hbm, emb_hbm, pos_hbm, out_hbm, idx_v, rows_v,
          idx_sem, fill_sem, gat_sem, out_sem):
    wid = lax.axis_index("s") * _NC + lax.axis_index("c")
    base = wid * _CHUNKS

    def issue_front(g, b):
        pltpu.async_copy(x_hbm.at[pl.ds((base + g) * _CS, _CS)],
                         idx_v.at[b], idx_sem.at[b])
        pltpu.async_copy(pos_hbm, rows_v.at[b], fill_sem.at[b])

    def wait_front(b):
        pltpu.make_async_copy(x_hbm.at[pl.ds(0, _CS)], idx_v.at[b],
                              idx_sem.at[b]).wait()
        pltpu.make_async_copy(pos_hbm, rows_v.at[b], fill_sem.at[b]).wait()

    def issue_gather(b):
        for c in range(_CS):
            for j in range(_NG):
                pltpu.async_copy(
                    emb_hbm.at[idx_v.at[b, c, j]],
                    rows_v.at[b, pl.ds((c * _NG + j) * _GCHUNK, _GCHUNK)],
                    gat_sem.at[b],
                    add=True,
                )

    def wait_gather(b):
        for _ in range(_CS * _NG):
            pltpu.make_async_copy(
                emb_hbm.at[idx_v.at[0, 0, 0]],
                rows_v.at[b, pl.ds(0, _GCHUNK)],
                gat_sem.at[b],
            ).wait()

    def issue_out(g, b):
        pltpu.async_copy(rows_v.at[b], out_hbm.at[base + g], out_sem.at[b])

    def wait_out(b):
        pltpu.make_async_copy(rows_v.at[b], out_hbm.at[0],
                              out_sem.at[b]).wait()

    issue_front(0, 0)
    issue_front(1, 1)

    def step(g, carry):
        b = lax.rem(g, _NBUF)
        wait_front(b)
        issue_gather(b)

        @pl.when(g >= 1)
        def _():
            b1 = lax.rem(g - 1, _NBUF)
            wait_gather(b1)
            issue_out(g - 1, b1)

        @pl.when(g >= 2)
        def _():
            wait_out(lax.rem(g - 2, _NBUF))

        @pl.when(g + 2 < _CHUNKS)
        def _():
            issue_front(g + 2, lax.rem(g + 2, _NBUF))

        return carry

    lax.fori_loop(0, _CHUNKS, step, 0)

    bl = (_CHUNKS - 1) % _NBUF
    wait_gather(bl)
    issue_out(_CHUNKS - 1, bl)
    wait_out((_CHUNKS - 2) % _NBUF)
    wait_out(bl)


@jax.jit
def _embed(x, emb, pos_emb):
    mesh = plsc.VectorSubcoreMesh(core_axis_name="c", subcore_axis_name="s")
    k = pl.kernel(
        _body,
        out_type=jax.ShapeDtypeStruct((_BATCH * _SEQ // _ROWS, _ROWS, _HID),
                                      jnp.float32),
        mesh=mesh,
        scratch_types=[
            pltpu.VMEM((_NBUF, _CS, _NG, _GCHUNK), jnp.int32),
            pltpu.VMEM((_NBUF, _ROWS, _HID), jnp.float32),
            pltpu.SemaphoreType.DMA((_NBUF,)),
            pltpu.SemaphoreType.DMA((_NBUF,)),
            pltpu.SemaphoreType.DMA((_NBUF,)),
            pltpu.SemaphoreType.DMA((_NBUF,)),
        ],
        compiler_params=pltpu.CompilerParams(use_tc_tiling_on_sc=False),
    )
    xr = x.reshape(_BATCH, _NG, _GCHUNK).astype(jnp.int32)
    pos2 = jnp.concatenate([pos_emb] * _CS, axis=0)
    out = k(xr, emb, pos2)
    return out.reshape(_BATCH, _SEQ, _HID)


def kernel(x, emb, pos_emb):
    return _embed(x, emb, pos_emb)
